# baseline (speedup 1.0000x reference)
import functools

import jax
import jax.numpy as jnp
from jax import lax
from jax.experimental import pallas as pl
from jax.experimental.pallas import tpu as pltpu
from jax.experimental.pallas import tpu_sc as plsc

N, P2, W2, CK, TOPK = 8, 49, 64, 192, 8
ROWS = N * P2 * TOPK
OPB = P2 * TOPK
NW = 32
WPB = NW // N
RPQ = 13
PAD = 512
LANES = 16
NOB = 4
BLK = 16
NBLK = OPB // BLK


def _sc_body(idx_hbm, w_hbm, k_hbm, out_hbm, idx_v, w_v,
             gref, ob0, ob1, ob2, ob3,
             gs0, gs1, ss0, ss1, ss2, ss3):
  obufs = (ob0, ob1, ob2, ob3)
  gsems = (gs0, gs1)
  ssems = (ss0, ss1, ss2, ss3)

  wid = lax.axis_index("s") * 2 + lax.axis_index("c")
  b = wid // WPB
  q = wid % WPB
  lo = q * 12
  tbase = b * P2
  obase = b * OPB

  pltpu.sync_copy(idx_hbm.at[pl.ds(b, 1)], idx_v)
  pltpu.sync_copy(w_hbm.at[pl.ds(b, 1)], w_v)

  def start_gather(m):
    src = k_hbm.at[pl.ds(tbase + lo + m, 1)]
    dst = gref.at[pl.ds(m % 2, 1)]
    for par in range(2):
      @pl.when(m % 2 == par)
      def _(_par=par):
        pltpu.make_async_copy(src, dst, gsems[_par]).start()

  def wait_gather(m):
    src = k_hbm.at[pl.ds(tbase + lo + m, 1)]
    dst = gref.at[pl.ds(m % 2, 1)]
    for par in range(2):
      @pl.when(m % 2 == par)
      def _(_par=par):
        pltpu.make_async_copy(src, dst, gsems[_par]).wait()

  def wait_scatter(s):
    pltpu.make_async_copy(obufs[s], out_hbm.at[pl.ds(obase, 1)],
                          ssems[s]).wait()

  def emit(o, s, match, wval, gbuf, flag):
    @pl.when(match & (flag == 1))
    def _():
      wait_scatter(s)

    @pl.when(match)
    def _():
      wvec = jnp.full((LANES,), wval, jnp.float32)
      ob = obufs[s]

      def mb(r, carry):
        for t in range(CK // LANES):
          sl = pl.ds(t * LANES, LANES)
          ob[0, r, sl] = gbuf[0, r, sl] * wvec
        return carry

      lax.fori_loop(0, W2, mb, 0)
      pltpu.make_async_copy(obufs[s], out_hbm.at[pl.ds(obase + o, 1)],
                            ssems[s]).start()

    return flag | match.astype(jnp.int32)

  start_gather(jnp.int32(0))
  flags = (jnp.int32(0),) * NOB

  def row_body(m, fl):
    @pl.when(m + 1 < RPQ)
    def _():
      start_gather(m + 1)

    wait_gather(m)
    gbuf = gref.at[pl.ds(m % 2, 1)]
    target = jnp.where(
        m < 12, lo + m,
        jnp.where(q == WPB - 1, jnp.int32(48), jnp.int32(1000)))

    def blk_body(bo, fl2):
      o0 = bo * BLK
      tvec = idx_v[0, pl.ds(o0, BLK)]
      wvec16 = w_v[0, pl.ds(o0, BLK)]
      for j in range(BLK):
        fl2 = tuple(
            emit(o0 + j, jj, (tvec[j] == target), wvec16[j], gbuf, f)
            if jj == (j % NOB) else f
            for jj, f in enumerate(fl2))
      return fl2

    fl = lax.fori_loop(0, NBLK, blk_body, fl)
    tvec = idx_v[0, pl.ds(NBLK * BLK, BLK)]
    wvec16 = w_v[0, pl.ds(NBLK * BLK, BLK)]
    for j in range(OPB - NBLK * BLK):
      s = j % NOB
      fl = tuple(
          emit(NBLK * BLK + j, jj, (tvec[j] == target), wvec16[j], gbuf, f)
          if jj == s else f
          for jj, f in enumerate(fl))
    return fl

  flags = lax.fori_loop(0, RPQ, row_body, flags)

  for s in range(NOB):
    @pl.when(flags[s] == 1)
    def _(_s=s):
      wait_scatter(_s)


_mesh = plsc.VectorSubcoreMesh(core_axis_name="c", subcore_axis_name="s")

_sc_call = functools.partial(
    pl.kernel,
    out_type=jax.ShapeDtypeStruct((ROWS, W2, CK), jnp.float32),
    mesh=_mesh,
    scratch_types=[
        pltpu.VMEM((1, PAD), jnp.int32),
        pltpu.VMEM((1, PAD), jnp.float32),
    ] + [pltpu.VMEM((2, W2, CK), jnp.float32)]
      + [pltpu.VMEM((1, W2, CK), jnp.float32)] * NOB
      + [pltpu.SemaphoreType.DMA] * (2 + NOB),
    compiler_params=pltpu.CompilerParams(use_tc_tiling_on_sc=True),
)(_sc_body)


def kernel(r_idx, r_weight, k):
  n, p2, w2, c_k = k.shape
  topk = r_idx.shape[-1]
  table = k.reshape(n * p2, w2, c_k)
  idx = jnp.pad(r_idx.reshape(n, OPB), ((0, 0), (0, PAD - OPB)))
  wgt = jnp.pad(r_weight.reshape(n, OPB), ((0, 0), (0, PAD - OPB)))
  out = _sc_call(idx, wgt, table)
  return out.reshape(n, p2, topk, w2, c_k)


# speedup vs baseline: 1.8628x; 1.8628x over previous
import functools

import jax
import jax.numpy as jnp
from jax import lax
from jax.experimental import pallas as pl
from jax.experimental.pallas import tpu as pltpu
from jax.experimental.pallas import tpu_sc as plsc

N, P2, W2, CK, TOPK = 8, 49, 64, 192, 8
ROWS = N * P2 * TOPK
OPB = P2 * TOPK
NW = 32
WPB = NW // N
RPQ = 13
HW2 = W2 // 2
PAD = 512
LANES = 16


def _sc_body(idx_hbm, w_hbm, k_hbm, out_hbm, idx_v, w_v,
             gref, ob0, ob1, gsem, ss0, ss1):
  obufs = (ob0, ob1)
  ssems = (ss0, ss1)

  wid = lax.axis_index("s") * 2 + lax.axis_index("c")
  b = wid // WPB
  q = wid % WPB
  lo = q * 12
  sz = jnp.where(q == WPB - 1, jnp.int32(RPQ), jnp.int32(12))
  tbase = b * P2
  obase = b * OPB

  pltpu.sync_copy(idx_hbm.at[pl.ds(b, 1)], idx_v)
  pltpu.sync_copy(w_hbm.at[pl.ds(b, 1)], w_v)

  state = (jnp.int32(0), jnp.int32(0), jnp.int32(0))

  for half in range(2):
    roff = half * HW2
    for m in range(RPQ):
      pltpu.make_async_copy(
          k_hbm.at[pl.ds(tbase + lo + m, 1), pl.ds(roff, HW2)],
          gref.at[pl.ds(m, 1)], gsem).start()
    for m in range(RPQ):
      pltpu.make_async_copy(
          k_hbm.at[pl.ds(tbase + lo + m, 1), pl.ds(roff, HW2)],
          gref.at[pl.ds(m, 1)], gsem).wait()

    def pos_body(o, st, _roff=roff):
      cur, f0, f1 = st
      t = idx_v[0, pl.ds(o, LANES)][0]
      wval = w_v[0, pl.ds(o, LANES)][0]
      m = t - lo
      inr = (m >= 0) & (m < sz)
      par = cur & 1
      gslab = gref.at[pl.ds(m, 1)]

      for p in range(2):
        pc = inr & (par == p)
        flag = f0 if p == 0 else f1

        @pl.when(pc & (flag == 1))
        def _(_p=p):
          pltpu.make_async_copy(
              obufs[_p], out_hbm.at[pl.ds(obase, 1), pl.ds(_roff, HW2)],
              ssems[_p]).wait()

        @pl.when(pc)
        def _(_p=p):
          wvec = jnp.full((LANES,), wval, jnp.float32)
          ob = obufs[_p]

          def mb(r, carry):
            for tt in range(CK // LANES):
              sl = pl.ds(tt * LANES, LANES)
              ob[0, r, sl] = gslab[0, r, sl] * wvec
            return carry

          lax.fori_loop(0, HW2, mb, 0)
          pltpu.make_async_copy(
              obufs[_p], out_hbm.at[pl.ds(obase + o, 1), pl.ds(_roff, HW2)],
              ssems[_p]).start()

      inri = inr.astype(jnp.int32)
      f0n = f0 | (inri & (par == 0).astype(jnp.int32))
      f1n = f1 | (inri & (par == 1).astype(jnp.int32))
      return (cur + inri, f0n, f1n)

    state = lax.fori_loop(0, OPB, pos_body, state)

  _, f0, f1 = state
  for p in range(2):
    flag = f0 if p == 0 else f1

    @pl.when(flag == 1)
    def _(_p=p):
      pltpu.make_async_copy(
          obufs[_p], out_hbm.at[pl.ds(obase, 1), pl.ds(0, HW2)],
          ssems[_p]).wait()


_mesh = plsc.VectorSubcoreMesh(core_axis_name="c", subcore_axis_name="s")

_sc_call = functools.partial(
    pl.kernel,
    out_type=jax.ShapeDtypeStruct((ROWS, W2, CK), jnp.float32),
    mesh=_mesh,
    scratch_types=[
        pltpu.VMEM((1, PAD), jnp.int32),
        pltpu.VMEM((1, PAD), jnp.float32),
        pltpu.VMEM((RPQ, HW2, CK), jnp.float32),
        pltpu.VMEM((1, HW2, CK), jnp.float32),
        pltpu.VMEM((1, HW2, CK), jnp.float32),
    ] + [pltpu.SemaphoreType.DMA] * 3,
    compiler_params=pltpu.CompilerParams(use_tc_tiling_on_sc=True),
)(_sc_body)


def kernel(r_idx, r_weight, k):
  n, p2, w2, c_k = k.shape
  topk = r_idx.shape[-1]
  table = k.reshape(n * p2, w2, c_k)
  idx = jnp.pad(r_idx.reshape(n, OPB), ((0, 0), (0, PAD - OPB)))
  wgt = jnp.pad(r_weight.reshape(n, OPB), ((0, 0), (0, PAD - OPB)))
  out = _sc_call(idx, wgt, table)
  return out.reshape(n, p2, topk, w2, c_k)


# speedup vs baseline: 5.7917x; 3.1091x over previous
import functools

import jax
import jax.numpy as jnp
from jax import lax
from jax.experimental import pallas as pl
from jax.experimental.pallas import tpu as pltpu
from jax.experimental.pallas import tpu_sc as plsc

N, P2, W2, CK, TOPK = 8, 49, 64, 192, 8
ROWS = N * P2 * TOPK
NW = 32
RPW = ROWS // NW
PAD = 128
LANES = 16
NBUF = 7
LOOK = NBUF - 2


def _sc_body(idx_hbm, w_hbm, k_hbm, out_hbm, idx_v, w_v, *scratch):
  bufs = scratch[:NBUF]
  gsems = scratch[NBUF:2 * NBUF]
  ssems = scratch[2 * NBUF:3 * NBUF]

  wid = lax.axis_index("s") * 2 + lax.axis_index("c")
  base = wid * RPW
  boff = (wid // 4) * P2

  pltpu.sync_copy(idx_hbm.at[pl.ds(wid, 1)], idx_v)
  pltpu.sync_copy(w_hbm.at[pl.ds(wid, 1)], w_v)

  def row_of(c):
    return idx_v[0, pl.ds(c, LANES)][0] + boff

  def start_gather(c, p):
    pltpu.make_async_copy(
        k_hbm.at[pl.ds(row_of(c), 1)], bufs[p], gsems[p]).start()

  def wait_gather(c, p):
    pltpu.make_async_copy(
        k_hbm.at[pl.ds(row_of(c), 1)], bufs[p], gsems[p]).wait()

  def start_scatter(c, p):
    pltpu.make_async_copy(
        bufs[p], out_hbm.at[pl.ds(base + c, 1)], ssems[p]).start()

  def wait_scatter(c, p):
    pltpu.make_async_copy(
        bufs[p], out_hbm.at[pl.ds(base + c, 1)], ssems[p]).wait()

  def do_mult(c, p):
    wvec = jnp.full((LANES,), w_v[0, pl.ds(c, LANES)][0], jnp.float32)
    buf = bufs[p]

    def mb(r, carry):
      for t in range(CK // LANES):
        sl = pl.ds(t * LANES, LANES)
        buf[0, r, sl] = buf[0, r, sl] * wvec
      return carry

    lax.fori_loop(0, W2, mb, 0)

  for c in range(LOOK):
    start_gather(c, c % NBUF)

  M = (RPW - LOOK) // NBUF

  def outer(o, carry):
    for par in range(NBUF):
      c = o * NBUF + par
      p = par
      q = (par + LOOK) % NBUF
      wait_gather(c, p)
      do_mult(c, p)
      start_scatter(c, p)

      @pl.when(c >= NBUF - LOOK)
      def _():
        wait_scatter(c - (NBUF - LOOK), q)

      start_gather(c + LOOK, q)
    return carry

  lax.fori_loop(0, M, outer, 0)

  for c in range(M * NBUF, RPW):
    p = c % NBUF
    wait_gather(c, p)
    do_mult(c, p)
    start_scatter(c, p)
    nxt = c + LOOK
    if nxt < RPW:
      q = nxt % NBUF
      wait_scatter(nxt - NBUF, q)
      start_gather(nxt, q)

  for c in range(RPW - NBUF, RPW):
    wait_scatter(c, c % NBUF)


_mesh = plsc.VectorSubcoreMesh(core_axis_name="c", subcore_axis_name="s")

_sc_call = functools.partial(
    pl.kernel,
    out_type=jax.ShapeDtypeStruct((ROWS, W2, CK), jnp.float32),
    mesh=_mesh,
    scratch_types=[
        pltpu.VMEM((1, PAD), jnp.int32),
        pltpu.VMEM((1, PAD), jnp.float32),
    ] + [pltpu.VMEM((1, W2, CK), jnp.float32)] * NBUF
      + [pltpu.SemaphoreType.DMA] * (2 * NBUF),
    compiler_params=pltpu.CompilerParams(use_tc_tiling_on_sc=True),
)(_sc_body)


def kernel(r_idx, r_weight, k):
  n, p2, w2, c_k = k.shape
  topk = r_idx.shape[-1]
  table = k.reshape(n * p2, w2, c_k)
  idx = jnp.pad(r_idx.reshape(NW, RPW), ((0, 0), (0, PAD - RPW)))
  wgt = jnp.pad(r_weight.reshape(NW, RPW), ((0, 0), (0, PAD - RPW)))
  out = _sc_call(idx, wgt, table)
  return out.reshape(n, p2, topk, w2, c_k)
